# baseline (speedup 1.0000x reference)
import functools

import jax
import jax.numpy as jnp
from jax import lax
from jax.experimental import pallas as pl
from jax.experimental.pallas import tpu as pltpu
from jax.experimental.pallas import tpu_sc as plsc

NC = 2
NS = 16
NW = NC * NS
LANES = 16

EDGE_K = 100


def _make_sc_scatter(n, h, e, nbuf, k=EDGE_K):
    c_chunks = e // (NW * k)
    assert e == NW * c_chunks * k and c_chunks % nbuf == 0
    n_out_subs = 10
    rows_per_out = n // n_out_subs
    zrows = 200
    assert rows_per_out % zrows == 0 and rows_per_out % 8 == 0
    mesh = plsc.VectorSubcoreMesh(
        core_axis_name="c", subcore_axis_name="s", num_cores=NC, num_subcores=NS
    )

    @functools.partial(
        pl.kernel,
        out_type=jax.ShapeDtypeStruct((NC, n, h), jnp.float32),
        mesh=mesh,
        compiler_params=pltpu.CompilerParams(use_tc_tiling_on_sc=False),
        scratch_types=[
            pltpu.VMEM((c_chunks, k), jnp.int32),
            pltpu.VMEM((c_chunks, k), jnp.int32),
            pltpu.VMEM((nbuf, k, h), jnp.float32),
            pltpu.VMEM((zrows, h), jnp.float32),
            pltpu.VMEM_SHARED((n, h), jnp.float32),
            pltpu.SemaphoreType.DMA((nbuf,)),
            pltpu.SemaphoreType.DMA((nbuf,)),
        ],
    )
    def scat(g_hbm, src_hbm, dst_hbm, out_hbm, src_v, dst_v, rows_v, zbuf, acc, sem_g, sem_s):
        cid = lax.axis_index("c")
        sid = lax.axis_index("s")
        wid = sid * NC + cid
        pltpu.sync_copy(src_hbm.at[wid], src_v)
        pltpu.sync_copy(dst_hbm.at[wid], dst_v)

        def zero_store(t, _):
            i = t // (h // LANES)
            j = t % (h // LANES)
            zbuf[i, pl.ds(j * LANES, LANES)] = jnp.zeros((LANES,), jnp.float32)
            return 0

        lax.fori_loop(0, zrows * (h // LANES), zero_store, 0)

        def zero_copy(t, _):
            pltpu.sync_copy(
                zbuf, acc.at[pl.ds(sid * rows_per_out + t * zrows, zrows)]
            )
            return 0

        @pl.when(sid < n_out_subs)
        def _():
            lax.fori_loop(0, rows_per_out // zrows, zero_copy, 0)

        plsc.subcore_barrier()

        n_groups = c_chunks // nbuf

        for b in range(nbuf):
            pltpu.async_copy(g_hbm.at[src_v.at[b]], rows_v.at[b], sem_g.at[b])

        def group(g, _):
            for b in range(nbuf):
                c = g * nbuf + b
                pltpu.make_async_copy(
                    g_hbm.at[src_v.at[c]], rows_v.at[b], sem_g.at[b]
                ).wait()
                pltpu.async_copy(
                    rows_v.at[b], acc.at[dst_v.at[c]], sem_s.at[b], add=True
                )

            @pl.when(g < n_groups - 1)
            def _():
                for b in range(nbuf):
                    c = g * nbuf + b
                    pltpu.make_async_copy(
                        rows_v.at[b], acc.at[dst_v.at[c]], sem_s.at[b]
                    ).wait()
                    pltpu.async_copy(
                        g_hbm.at[src_v.at[c + nbuf]], rows_v.at[b], sem_g.at[b]
                    )

            return 0

        lax.fori_loop(0, n_groups, group, 0)
        for b in range(nbuf):
            c = (n_groups - 1) * nbuf + b
            pltpu.make_async_copy(
                rows_v.at[b], acc.at[dst_v.at[c]], sem_s.at[b]
            ).wait()
        plsc.subcore_barrier()

        @pl.when(sid < n_out_subs)
        def _():
            pltpu.sync_copy(
                acc.at[pl.ds(sid * rows_per_out, rows_per_out)],
                out_hbm.at[cid, pl.ds(sid * rows_per_out, rows_per_out)],
            )

    return scat


def _make_sc_degree(n, e):
    h = LANES
    k = EDGE_K
    c_chunks = e // (NW * k)
    assert e == NW * c_chunks * k
    n_out_subs = 10
    rows_per_out = n // n_out_subs
    zrows = 200
    group = 10
    assert c_chunks % group == 0
    mesh = plsc.VectorSubcoreMesh(
        core_axis_name="c", subcore_axis_name="s", num_cores=NC, num_subcores=NS
    )

    @functools.partial(
        pl.kernel,
        out_type=jax.ShapeDtypeStruct((NC, n, 64), jnp.float32),
        mesh=mesh,
        compiler_params=pltpu.CompilerParams(use_tc_tiling_on_sc=False),
        scratch_types=[
            pltpu.VMEM((c_chunks, k), jnp.int32),
            pltpu.VMEM((k, h), jnp.float32),
            pltpu.VMEM((zrows, h), jnp.float32),
            pltpu.VMEM((rows_per_out, h), jnp.float32),
            pltpu.VMEM((rows_per_out, 64), jnp.float32),
            pltpu.VMEM_SHARED((n, h), jnp.float32),
            pltpu.SemaphoreType.DMA,
        ],
    )
    def deg(dst_hbm, out_hbm, dst_v, ones_v, zbuf, rep_in, rep_out, acc, sem):
        cid = lax.axis_index("c")
        sid = lax.axis_index("s")
        wid = sid * NC + cid
        pltpu.sync_copy(dst_hbm.at[wid], dst_v)

        def fill(t, _):
            i = t // (h // LANES)
            j = t % (h // LANES)
            ones_v[i, pl.ds(j * LANES, LANES)] = jnp.ones((LANES,), jnp.float32)
            return 0

        lax.fori_loop(0, k * (h // LANES), fill, 0)

        def zero_store(t, _):
            i = t // (h // LANES)
            j = t % (h // LANES)
            zbuf[i, pl.ds(j * LANES, LANES)] = jnp.zeros((LANES,), jnp.float32)
            return 0

        lax.fori_loop(0, zrows * (h // LANES), zero_store, 0)

        def zero_copy(t, _):
            pltpu.sync_copy(
                zbuf, acc.at[pl.ds(sid * rows_per_out + t * zrows, zrows)]
            )
            return 0

        @pl.when(sid < n_out_subs)
        def _():
            lax.fori_loop(0, rows_per_out // zrows, zero_copy, 0)

        plsc.subcore_barrier()

        def body(gi, _):
            for j in range(group):
                pltpu.async_copy(
                    ones_v, acc.at[dst_v.at[gi * group + j]], sem, add=True
                )
            for j in range(group):
                pltpu.make_async_copy(
                    ones_v, acc.at[dst_v.at[gi * group + j]], sem
                ).wait()
            return 0

        lax.fori_loop(0, c_chunks // group, body, 0)
        plsc.subcore_barrier()

        rps = n // NS
        pltpu.sync_copy(acc.at[pl.ds(sid * rps, rps)], rep_in.at[pl.ds(0, rps)])

        def rep(t, _):
            v = rep_in[t]
            for j in range(64 // h):
                rep_out[t, pl.ds(j * h, h)] = v
            return 0

        lax.fori_loop(0, rps, rep, 0)
        pltpu.sync_copy(
            rep_out.at[pl.ds(0, rps)], out_hbm.at[cid, pl.ds(sid * rps, rps)]
        )

    return deg


def _make_a0(npk, dpk, rb):
    def body(x, w, h_o):
        h_o[...] = jnp.dot(x[...], w[...], preferred_element_type=jnp.float32)

    return pl.pallas_call(
        body,
        grid=(npk // rb,),
        in_specs=[
            pl.BlockSpec((rb, dpk), lambda i: (i, 0)),
            pl.BlockSpec((dpk, 128), lambda i: (0, 0)),
        ],
        out_specs=pl.BlockSpec((rb, 128), lambda i: (i, 0)),
        out_shape=jax.ShapeDtypeStruct((npk, 128), jnp.float32),
    )


def _make_a1(npk, rb):
    def body(degp, h, dis_o, g_o):
        dis = lax.rsqrt(degp[0] + degp[1] + 1.0)
        dis_o[...] = dis
        g_o[...] = h[...] * dis

    return pl.pallas_call(
        body,
        grid=(npk // rb,),
        in_specs=[
            pl.BlockSpec((2, rb, 128), lambda i: (0, i, 0)),
            pl.BlockSpec((rb, 128), lambda i: (i, 0)),
        ],
        out_specs=[
            pl.BlockSpec((rb, 128), lambda i: (i, 0)),
            pl.BlockSpec((rb, 128), lambda i: (i, 0)),
        ],
        out_shape=[
            jax.ShapeDtypeStruct((npk, 128), jnp.float32),
            jax.ShapeDtypeStruct((npk, 128), jnp.float32),
        ],
    )


def _make_combine_matmul(npk, hout, rb, narrow_g):
    def body(dis, hprev, p, b, w, h_o, g_o):
        dd = dis[...]
        z = dd * (p[0] + p[1]) + (dd * dd) * hprev[...] + b[...]
        z = jnp.maximum(z, 0.0)
        hh = jnp.dot(z, w[...], preferred_element_type=jnp.float32)
        if narrow_g:
            dn = jnp.concatenate([dd[:, 0:16], dd[:, 64:80]], axis=1)
        else:
            dn = dd
        h_o[...] = hh
        g_o[...] = hh * dn

    return pl.pallas_call(
        body,
        grid=(npk // rb,),
        in_specs=[
            pl.BlockSpec((rb, 128), lambda i: (i, 0)),
            pl.BlockSpec((rb, 128), lambda i: (i, 0)),
            pl.BlockSpec((2, rb, 128), lambda i: (0, i, 0)),
            pl.BlockSpec((1, 128), lambda i: (0, 0)),
            pl.BlockSpec((128, hout), lambda i: (0, 0)),
        ],
        out_specs=[
            pl.BlockSpec((rb, hout), lambda i: (i, 0)),
            pl.BlockSpec((rb, hout), lambda i: (i, 0)),
        ],
        out_shape=[
            jax.ShapeDtypeStruct((npk, hout), jnp.float32),
            jax.ShapeDtypeStruct((npk, hout), jnp.float32),
        ],
    )


def _make_final(npk, o, rb):
    def body(dis, h3, p, b, out_o):
        dn = jnp.concatenate([dis[:, 0:16], dis[:, 64:80]], axis=1)
        full = dn * (p[0] + p[1]) + (dn * dn) * h3[...] + b[...]
        outs = []
        for half in range(2):
            o2 = full[:, 16 * half : 16 * half + o]
            mx = jnp.max(o2, axis=1, keepdims=True)
            lse = mx + jnp.log(jnp.sum(jnp.exp(o2 - mx), axis=1, keepdims=True))
            outs.append(o2 - lse)
        out_o[...] = jnp.concatenate(outs, axis=1)

    return pl.pallas_call(
        body,
        grid=(npk // rb,),
        in_specs=[
            pl.BlockSpec((rb, 128), lambda i: (i, 0)),
            pl.BlockSpec((rb, 32), lambda i: (i, 0)),
            pl.BlockSpec((2, rb, 32), lambda i: (0, i, 0)),
            pl.BlockSpec((1, 32), lambda i: (0, 0)),
        ],
        out_specs=pl.BlockSpec((rb, 2 * o), lambda i: (i, 0)),
        out_shape=jax.ShapeDtypeStruct((npk, 2 * o), jnp.float32),
    )


def _blockdiag(w):
    z = jnp.zeros_like(w)
    return jnp.concatenate(
        [jnp.concatenate([w, z], axis=1), jnp.concatenate([z, w], axis=1)], axis=0
    )


def kernel(x, edge_index, W1, b1, W2, b2, W3, b3):
    n, d = x.shape
    hdim = W1.shape[1]
    o = W3.shape[1]
    e = edge_index.shape[1]
    npk = n // 2
    rb = 1000

    k = EDGE_K
    c_chunks = e // (NW * k)
    src3 = edge_index[0].reshape(NW, c_chunks, k)
    dst3 = edge_index[1].reshape(NW, c_chunks, k)
    k64 = 50
    c64 = e // (NW * k64)
    src3b = edge_index[0].reshape(NW, c64, k64)
    dst3b = edge_index[1].reshape(NW, c64, k64)

    w1d = _blockdiag(W1)
    w2d = _blockdiag(W2)
    w3p = jnp.zeros((hdim, LANES), jnp.float32).at[:, :o].set(W3)
    w3d = _blockdiag(w3p)
    b1p = jnp.concatenate([b1, b1]).reshape(1, 2 * hdim)
    b2p = jnp.concatenate([b2, b2]).reshape(1, 2 * hdim)
    b3pad = jnp.zeros((LANES,), jnp.float32).at[:o].set(b3)
    b3p = jnp.concatenate([b3pad, b3pad]).reshape(1, 2 * LANES)

    xpair = x.reshape(npk, 2 * d)

    sc16 = _make_sc_scatter(n, LANES, e, nbuf=10)
    sc64 = _make_sc_scatter(n, hdim, e, nbuf=10, k=k64)

    deg64 = _make_sc_degree(n, e)(dst3)
    h1p = _make_a0(npk, 2 * d, rb)(xpair, w1d)
    dis64p, g1p = _make_a1(npk, rb)(deg64.reshape(2, npk, 128), h1p)
    p1 = sc64(g1p.reshape(n, hdim), src3b, dst3b)
    h2p, g2p = _make_combine_matmul(npk, 128, rb, False)(
        dis64p, h1p, p1.reshape(2, npk, 128), b1p, w2d
    )
    p2 = sc64(g2p.reshape(n, hdim), src3b, dst3b)
    h3pk, g3pk = _make_combine_matmul(npk, 32, rb, True)(
        dis64p, h2p, p2.reshape(2, npk, 128), b2p, w3d
    )
    p3 = sc16(g3pk.reshape(n, LANES), src3, dst3)
    outp = _make_final(npk, o, rb)(dis64p, h3pk, p3.reshape(2, npk, 32), b3p)
    return outp.reshape(n, o)


# speedup vs baseline: 1.1014x; 1.1014x over previous
import functools

import jax
import jax.numpy as jnp
from jax import lax
from jax.experimental import pallas as pl
from jax.experimental.pallas import tpu as pltpu
from jax.experimental.pallas import tpu_sc as plsc

NC = 2
NS = 16
NW = NC * NS
LANES = 16

EDGE_K = 100


def _make_sc_scatter(n, h, e, nbuf, k=EDGE_K):
    c_chunks = e // (NW * k)
    assert e == NW * c_chunks * k and c_chunks % nbuf == 0
    n_out_subs = 10
    rows_per_out = n // n_out_subs
    zrows = 200
    assert rows_per_out % zrows == 0 and rows_per_out % 8 == 0
    mesh = plsc.VectorSubcoreMesh(
        core_axis_name="c", subcore_axis_name="s", num_cores=NC, num_subcores=NS
    )

    @functools.partial(
        pl.kernel,
        out_type=jax.ShapeDtypeStruct((NC, n, h), jnp.float32),
        mesh=mesh,
        compiler_params=pltpu.CompilerParams(use_tc_tiling_on_sc=False),
        scratch_types=[
            pltpu.VMEM((c_chunks, k), jnp.int32),
            pltpu.VMEM((c_chunks, k), jnp.int32),
            pltpu.VMEM((nbuf, k, h), jnp.float32),
            pltpu.VMEM((zrows, h), jnp.float32),
            pltpu.VMEM_SHARED((n, h), jnp.float32),
            pltpu.SemaphoreType.DMA((nbuf,)),
            pltpu.SemaphoreType.DMA((nbuf,)),
        ],
    )
    def scat(g_hbm, src_hbm, dst_hbm, out_hbm, src_v, dst_v, rows_v, zbuf, acc, sem_g, sem_s):
        cid = lax.axis_index("c")
        sid = lax.axis_index("s")
        wid = sid * NC + cid
        pltpu.sync_copy(src_hbm.at[wid], src_v)
        pltpu.sync_copy(dst_hbm.at[wid], dst_v)

        def zero_store(t, _):
            i = t // (h // LANES)
            j = t % (h // LANES)
            zbuf[i, pl.ds(j * LANES, LANES)] = jnp.zeros((LANES,), jnp.float32)
            return 0

        lax.fori_loop(0, zrows * (h // LANES), zero_store, 0)

        def zero_copy(t, _):
            pltpu.sync_copy(
                zbuf, acc.at[pl.ds(sid * rows_per_out + t * zrows, zrows)]
            )
            return 0

        @pl.when(sid < n_out_subs)
        def _():
            lax.fori_loop(0, rows_per_out // zrows, zero_copy, 0)

        plsc.subcore_barrier()

        n_groups = c_chunks // nbuf

        for b in range(nbuf):
            pltpu.async_copy(g_hbm.at[src_v.at[b]], rows_v.at[b], sem_g.at[b])

        def group(g, _):
            for b in range(nbuf):
                c = g * nbuf + b
                pltpu.make_async_copy(
                    g_hbm.at[src_v.at[c]], rows_v.at[b], sem_g.at[b]
                ).wait()
                pltpu.async_copy(
                    rows_v.at[b], acc.at[dst_v.at[c]], sem_s.at[b], add=True
                )

            @pl.when(g < n_groups - 1)
            def _():
                for b in range(nbuf):
                    c = g * nbuf + b
                    pltpu.make_async_copy(
                        rows_v.at[b], acc.at[dst_v.at[c]], sem_s.at[b]
                    ).wait()
                    pltpu.async_copy(
                        g_hbm.at[src_v.at[c + nbuf]], rows_v.at[b], sem_g.at[b]
                    )

            return 0

        lax.fori_loop(0, n_groups, group, 0)
        for b in range(nbuf):
            c = (n_groups - 1) * nbuf + b
            pltpu.make_async_copy(
                rows_v.at[b], acc.at[dst_v.at[c]], sem_s.at[b]
            ).wait()
        plsc.subcore_barrier()

        @pl.when(sid < n_out_subs)
        def _():
            pltpu.sync_copy(
                acc.at[pl.ds(sid * rows_per_out, rows_per_out)],
                out_hbm.at[cid, pl.ds(sid * rows_per_out, rows_per_out)],
            )

    return scat


def _make_sc_degree(n, e):
    h = LANES
    k = EDGE_K
    c_chunks = e // (NW * k)
    assert e == NW * c_chunks * k
    n_out_subs = 10
    rows_per_out = n // n_out_subs
    zrows = 200
    group = 10
    assert c_chunks % group == 0
    mesh = plsc.VectorSubcoreMesh(
        core_axis_name="c", subcore_axis_name="s", num_cores=NC, num_subcores=NS
    )

    @functools.partial(
        pl.kernel,
        out_type=jax.ShapeDtypeStruct((NC, n, 64), jnp.float32),
        mesh=mesh,
        compiler_params=pltpu.CompilerParams(use_tc_tiling_on_sc=False),
        scratch_types=[
            pltpu.VMEM((c_chunks, k), jnp.int32),
            pltpu.VMEM((k, h), jnp.float32),
            pltpu.VMEM((zrows, h), jnp.float32),
            pltpu.VMEM((rows_per_out, h), jnp.float32),
            pltpu.VMEM((rows_per_out, 64), jnp.float32),
            pltpu.VMEM_SHARED((n, h), jnp.float32),
            pltpu.SemaphoreType.DMA,
        ],
    )
    def deg(dst_hbm, out_hbm, dst_v, ones_v, zbuf, rep_in, rep_out, acc, sem):
        cid = lax.axis_index("c")
        sid = lax.axis_index("s")
        wid = sid * NC + cid
        pltpu.sync_copy(dst_hbm.at[wid], dst_v)

        def fill(t, _):
            i = t // (h // LANES)
            j = t % (h // LANES)
            ones_v[i, pl.ds(j * LANES, LANES)] = jnp.ones((LANES,), jnp.float32)
            return 0

        lax.fori_loop(0, k * (h // LANES), fill, 0)

        def zero_store(t, _):
            i = t // (h // LANES)
            j = t % (h // LANES)
            zbuf[i, pl.ds(j * LANES, LANES)] = jnp.zeros((LANES,), jnp.float32)
            return 0

        lax.fori_loop(0, zrows * (h // LANES), zero_store, 0)

        def zero_copy(t, _):
            pltpu.sync_copy(
                zbuf, acc.at[pl.ds(sid * rows_per_out + t * zrows, zrows)]
            )
            return 0

        @pl.when(sid < n_out_subs)
        def _():
            lax.fori_loop(0, rows_per_out // zrows, zero_copy, 0)

        plsc.subcore_barrier()

        def body(gi, _):
            for j in range(group):
                pltpu.async_copy(
                    ones_v, acc.at[dst_v.at[gi * group + j]], sem, add=True
                )
            for j in range(group):
                pltpu.make_async_copy(
                    ones_v, acc.at[dst_v.at[gi * group + j]], sem
                ).wait()
            return 0

        lax.fori_loop(0, c_chunks // group, body, 0)
        plsc.subcore_barrier()

        rps = n // NS
        pltpu.sync_copy(acc.at[pl.ds(sid * rps, rps)], rep_in.at[pl.ds(0, rps)])

        def rep(t, _):
            v = rep_in[t]
            for j in range(64 // h):
                rep_out[t, pl.ds(j * h, h)] = v
            return 0

        lax.fori_loop(0, rps, rep, 0)
        pltpu.sync_copy(
            rep_out.at[pl.ds(0, rps)], out_hbm.at[cid, pl.ds(sid * rps, rps)]
        )

    return deg


def _make_a0(npk, dpk, rb):
    def body(x, w, h_o):
        h_o[...] = jnp.dot(x[...], w[...], preferred_element_type=jnp.float32)

    return pl.pallas_call(
        body,
        grid=(npk // rb,),
        in_specs=[
            pl.BlockSpec((rb, dpk), lambda i: (i, 0)),
            pl.BlockSpec((dpk, 128), lambda i: (0, 0)),
        ],
        out_specs=pl.BlockSpec((rb, 128), lambda i: (i, 0)),
        out_shape=jax.ShapeDtypeStruct((npk, 128), jnp.float32),
    )


def _make_a1(npk, rb):
    def body(degp, h, dis_o, g_o):
        dis = lax.rsqrt(degp[0] + degp[1] + 1.0)
        dis_o[...] = dis
        g_o[...] = h[...] * dis

    return pl.pallas_call(
        body,
        grid=(npk // rb,),
        in_specs=[
            pl.BlockSpec((2, rb, 128), lambda i: (0, i, 0)),
            pl.BlockSpec((rb, 128), lambda i: (i, 0)),
        ],
        out_specs=[
            pl.BlockSpec((rb, 128), lambda i: (i, 0)),
            pl.BlockSpec((rb, 128), lambda i: (i, 0)),
        ],
        out_shape=[
            jax.ShapeDtypeStruct((npk, 128), jnp.float32),
            jax.ShapeDtypeStruct((npk, 128), jnp.float32),
        ],
    )


def _make_combine_matmul(npk, hout, rb, narrow_g):
    def body(dis, hprev, p, b, w, h_o, g_o):
        dd = dis[...]
        z = dd * (p[0] + p[1]) + (dd * dd) * hprev[...] + b[...]
        z = jnp.maximum(z, 0.0)
        hh = jnp.dot(z, w[...], preferred_element_type=jnp.float32)
        if narrow_g:
            dn = jnp.concatenate([dd[:, 0:16], dd[:, 64:80]], axis=1)
        else:
            dn = dd
        h_o[...] = hh
        g_o[...] = hh * dn

    return pl.pallas_call(
        body,
        grid=(npk // rb,),
        in_specs=[
            pl.BlockSpec((rb, 128), lambda i: (i, 0)),
            pl.BlockSpec((rb, 128), lambda i: (i, 0)),
            pl.BlockSpec((2, rb, 128), lambda i: (0, i, 0)),
            pl.BlockSpec((1, 128), lambda i: (0, 0)),
            pl.BlockSpec((128, hout), lambda i: (0, 0)),
        ],
        out_specs=[
            pl.BlockSpec((rb, hout), lambda i: (i, 0)),
            pl.BlockSpec((rb, hout), lambda i: (i, 0)),
        ],
        out_shape=[
            jax.ShapeDtypeStruct((npk, hout), jnp.float32),
            jax.ShapeDtypeStruct((npk, hout), jnp.float32),
        ],
    )


def _make_final(npk, o, rb):
    def body(dis, h3, p, b, out_o):
        dn = jnp.concatenate([dis[:, 0:16], dis[:, 64:80]], axis=1)
        full = dn * (p[0] + p[1]) + (dn * dn) * h3[...] + b[...]
        outs = []
        for half in range(2):
            o2 = full[:, 16 * half : 16 * half + o]
            mx = jnp.max(o2, axis=1, keepdims=True)
            lse = mx + jnp.log(jnp.sum(jnp.exp(o2 - mx), axis=1, keepdims=True))
            outs.append(o2 - lse)
        out_o[...] = jnp.concatenate(outs, axis=1)

    return pl.pallas_call(
        body,
        grid=(npk // rb,),
        in_specs=[
            pl.BlockSpec((rb, 128), lambda i: (i, 0)),
            pl.BlockSpec((rb, 32), lambda i: (i, 0)),
            pl.BlockSpec((2, rb, 32), lambda i: (0, i, 0)),
            pl.BlockSpec((1, 32), lambda i: (0, 0)),
        ],
        out_specs=pl.BlockSpec((rb, 2 * o), lambda i: (i, 0)),
        out_shape=jax.ShapeDtypeStruct((npk, 2 * o), jnp.float32),
    )


def _blockdiag(w):
    z = jnp.zeros_like(w)
    return jnp.concatenate(
        [jnp.concatenate([w, z], axis=1), jnp.concatenate([z, w], axis=1)], axis=0
    )


def kernel(x, edge_index, W1, b1, W2, b2, W3, b3):
    n, d = x.shape
    hdim = W1.shape[1]
    o = W3.shape[1]
    e = edge_index.shape[1]
    npk = n // 2
    rb = 1000

    k = EDGE_K
    c_chunks = e // (NW * k)
    src3 = edge_index[0].reshape(NW, c_chunks, k)
    dst3 = edge_index[1].reshape(NW, c_chunks, k)

    w1d = _blockdiag(W1)
    w2d = _blockdiag(W2)
    w3p = jnp.zeros((hdim, LANES), jnp.float32).at[:, :o].set(W3)
    w3d = _blockdiag(w3p)
    b1p = jnp.concatenate([b1, b1]).reshape(1, 2 * hdim)
    b2p = jnp.concatenate([b2, b2]).reshape(1, 2 * hdim)
    b3pad = jnp.zeros((LANES,), jnp.float32).at[:o].set(b3)
    b3p = jnp.concatenate([b3pad, b3pad]).reshape(1, 2 * LANES)

    xpair = x.reshape(npk, 2 * d)

    sc16 = _make_sc_scatter(n, LANES, e, nbuf=10)
    sc64 = _make_sc_scatter(n, hdim, e, nbuf=5)

    deg64 = _make_sc_degree(n, e)(dst3)
    h1p = _make_a0(npk, 2 * d, rb)(xpair, w1d)
    dis64p, g1p = _make_a1(npk, rb)(deg64.reshape(2, npk, 128), h1p)
    p1 = sc64(g1p.reshape(n, hdim), src3, dst3)
    h2p, g2p = _make_combine_matmul(npk, 128, rb, False)(
        dis64p, h1p, p1.reshape(2, npk, 128), b1p, w2d
    )
    p2 = sc64(g2p.reshape(n, hdim), src3, dst3)
    h3pk, g3pk = _make_combine_matmul(npk, 32, rb, True)(
        dis64p, h2p, p2.reshape(2, npk, 128), b2p, w3d
    )
    p3 = sc16(g3pk.reshape(n, LANES), src3, dst3)
    outp = _make_final(npk, o, rb)(dis64p, h3pk, p3.reshape(2, npk, 32), b3p)
    return outp.reshape(n, o)


# speedup vs baseline: 1.1301x; 1.0261x over previous
import functools

import jax
import jax.numpy as jnp
from jax import lax
from jax.experimental import pallas as pl
from jax.experimental.pallas import tpu as pltpu
from jax.experimental.pallas import tpu_sc as plsc

NC = 2
NS = 16
NW = NC * NS
LANES = 16

EDGE_K = 100


def _make_sc_scatter(n, h, e, nbuf, k=EDGE_K):
    c_chunks = e // (NW * k)
    assert e == NW * c_chunks * k
    n_out_subs = 10
    rows_per_out = n // n_out_subs
    zrows = 200
    assert rows_per_out % zrows == 0 and rows_per_out % 8 == 0
    mesh = plsc.VectorSubcoreMesh(
        core_axis_name="c", subcore_axis_name="s", num_cores=NC, num_subcores=NS
    )

    @functools.partial(
        pl.kernel,
        out_type=jax.ShapeDtypeStruct((NC, n, h), jnp.float32),
        mesh=mesh,
        compiler_params=pltpu.CompilerParams(use_tc_tiling_on_sc=False),
        scratch_types=[
            pltpu.VMEM((c_chunks, k), jnp.int32),
            pltpu.VMEM((c_chunks, k), jnp.int32),
            pltpu.VMEM((nbuf, k, h), jnp.float32),
            pltpu.VMEM((zrows, h), jnp.float32),
            pltpu.VMEM_SHARED((n, h), jnp.float32),
            pltpu.SemaphoreType.DMA((nbuf,)),
            pltpu.SemaphoreType.DMA((nbuf,)),
        ],
    )
    def scat(g_hbm, src_hbm, dst_hbm, out_hbm, src_v, dst_v, rows_v, zbuf, acc, sem_g, sem_s):
        cid = lax.axis_index("c")
        sid = lax.axis_index("s")
        wid = sid * NC + cid
        pltpu.sync_copy(src_hbm.at[wid], src_v)
        pltpu.sync_copy(dst_hbm.at[wid], dst_v)

        def zero_store(t, _):
            i = t // (h // LANES)
            j = t % (h // LANES)
            zbuf[i, pl.ds(j * LANES, LANES)] = jnp.zeros((LANES,), jnp.float32)
            return 0

        lax.fori_loop(0, zrows * (h // LANES), zero_store, 0)

        def zero_copy(t, _):
            pltpu.sync_copy(
                zbuf, acc.at[pl.ds(sid * rows_per_out + t * zrows, zrows)]
            )
            return 0

        @pl.when(sid < n_out_subs)
        def _():
            lax.fori_loop(0, rows_per_out // zrows, zero_copy, 0)

        plsc.subcore_barrier()

        n_groups = (c_chunks + nbuf - 1) // nbuf

        for b in range(min(nbuf, c_chunks)):
            pltpu.async_copy(g_hbm.at[src_v.at[b]], rows_v.at[b], sem_g.at[b])

        def group(g, _):
            for b in range(nbuf):
                c = g * nbuf + b

                @pl.when(c < c_chunks)
                def _():
                    pltpu.make_async_copy(
                        g_hbm.at[src_v.at[c]], rows_v.at[b], sem_g.at[b]
                    ).wait()
                    pltpu.async_copy(
                        rows_v.at[b], acc.at[dst_v.at[c]], sem_s.at[b], add=True
                    )

            for b in range(nbuf):
                c = g * nbuf + b

                @pl.when(c + nbuf < c_chunks)
                def _():
                    pltpu.make_async_copy(
                        rows_v.at[b], acc.at[dst_v.at[c]], sem_s.at[b]
                    ).wait()
                    pltpu.async_copy(
                        g_hbm.at[src_v.at[c + nbuf]], rows_v.at[b], sem_g.at[b]
                    )

            return 0

        lax.fori_loop(0, n_groups, group, 0)
        for b in range(min(nbuf, c_chunks)):
            c = c_chunks - 1 - ((c_chunks - 1 - b) % nbuf)
            pltpu.make_async_copy(
                rows_v.at[b], acc.at[dst_v.at[c]], sem_s.at[b]
            ).wait()
        plsc.subcore_barrier()

        @pl.when(sid < n_out_subs)
        def _():
            pltpu.sync_copy(
                acc.at[pl.ds(sid * rows_per_out, rows_per_out)],
                out_hbm.at[cid, pl.ds(sid * rows_per_out, rows_per_out)],
            )

    return scat


def _make_sc_degree(n, e):
    h = LANES
    k = EDGE_K
    c_chunks = e // (NW * k)
    assert e == NW * c_chunks * k
    n_out_subs = 10
    rows_per_out = n // n_out_subs
    zrows = 200
    group = 10
    assert c_chunks % group == 0
    mesh = plsc.VectorSubcoreMesh(
        core_axis_name="c", subcore_axis_name="s", num_cores=NC, num_subcores=NS
    )

    @functools.partial(
        pl.kernel,
        out_type=jax.ShapeDtypeStruct((NC, n, 64), jnp.float32),
        mesh=mesh,
        compiler_params=pltpu.CompilerParams(use_tc_tiling_on_sc=False),
        scratch_types=[
            pltpu.VMEM((c_chunks, k), jnp.int32),
            pltpu.VMEM((k, h), jnp.float32),
            pltpu.VMEM((zrows, h), jnp.float32),
            pltpu.VMEM((rows_per_out, h), jnp.float32),
            pltpu.VMEM((rows_per_out, 64), jnp.float32),
            pltpu.VMEM_SHARED((n, h), jnp.float32),
            pltpu.SemaphoreType.DMA,
        ],
    )
    def deg(dst_hbm, out_hbm, dst_v, ones_v, zbuf, rep_in, rep_out, acc, sem):
        cid = lax.axis_index("c")
        sid = lax.axis_index("s")
        wid = sid * NC + cid
        pltpu.sync_copy(dst_hbm.at[wid], dst_v)

        def fill(t, _):
            i = t // (h // LANES)
            j = t % (h // LANES)
            ones_v[i, pl.ds(j * LANES, LANES)] = jnp.ones((LANES,), jnp.float32)
            return 0

        lax.fori_loop(0, k * (h // LANES), fill, 0)

        def zero_store(t, _):
            i = t // (h // LANES)
            j = t % (h // LANES)
            zbuf[i, pl.ds(j * LANES, LANES)] = jnp.zeros((LANES,), jnp.float32)
            return 0

        lax.fori_loop(0, zrows * (h // LANES), zero_store, 0)

        def zero_copy(t, _):
            pltpu.sync_copy(
                zbuf, acc.at[pl.ds(sid * rows_per_out + t * zrows, zrows)]
            )
            return 0

        @pl.when(sid < n_out_subs)
        def _():
            lax.fori_loop(0, rows_per_out // zrows, zero_copy, 0)

        plsc.subcore_barrier()

        def body(gi, _):
            for j in range(group):
                pltpu.async_copy(
                    ones_v, acc.at[dst_v.at[gi * group + j]], sem, add=True
                )
            for j in range(group):
                pltpu.make_async_copy(
                    ones_v, acc.at[dst_v.at[gi * group + j]], sem
                ).wait()
            return 0

        lax.fori_loop(0, c_chunks // group, body, 0)
        plsc.subcore_barrier()

        rps = n // NS
        pltpu.sync_copy(acc.at[pl.ds(sid * rps, rps)], rep_in.at[pl.ds(0, rps)])

        def rep(t, _):
            v = rep_in[t]
            for j in range(64 // h):
                rep_out[t, pl.ds(j * h, h)] = v
            return 0

        lax.fori_loop(0, rps, rep, 0)
        pltpu.sync_copy(
            rep_out.at[pl.ds(0, rps)], out_hbm.at[cid, pl.ds(sid * rps, rps)]
        )

    return deg


def _make_a0(npk, dpk, rb):
    def body(x, w, h_o):
        h_o[...] = jnp.dot(x[...], w[...], preferred_element_type=jnp.float32)

    return pl.pallas_call(
        body,
        grid=(npk // rb,),
        in_specs=[
            pl.BlockSpec((rb, dpk), lambda i: (i, 0)),
            pl.BlockSpec((dpk, 128), lambda i: (0, 0)),
        ],
        out_specs=pl.BlockSpec((rb, 128), lambda i: (i, 0)),
        out_shape=jax.ShapeDtypeStruct((npk, 128), jnp.float32),
    )


def _make_a1(npk, rb):
    def body(degp, h, dis_o, g_o):
        dis = lax.rsqrt(degp[0] + degp[1] + 1.0)
        dis_o[...] = dis
        g_o[...] = h[...] * dis

    return pl.pallas_call(
        body,
        grid=(npk // rb,),
        in_specs=[
            pl.BlockSpec((2, rb, 128), lambda i: (0, i, 0)),
            pl.BlockSpec((rb, 128), lambda i: (i, 0)),
        ],
        out_specs=[
            pl.BlockSpec((rb, 128), lambda i: (i, 0)),
            pl.BlockSpec((rb, 128), lambda i: (i, 0)),
        ],
        out_shape=[
            jax.ShapeDtypeStruct((npk, 128), jnp.float32),
            jax.ShapeDtypeStruct((npk, 128), jnp.float32),
        ],
    )


def _make_combine_matmul(npk, hout, rb, narrow_g):
    def body(dis, hprev, p, b, w, h_o, g_o):
        dd = dis[...]
        z = dd * (p[0] + p[1]) + (dd * dd) * hprev[...] + b[...]
        z = jnp.maximum(z, 0.0)
        hh = jnp.dot(z, w[...], preferred_element_type=jnp.float32)
        if narrow_g:
            dn = jnp.concatenate([dd[:, 0:16], dd[:, 64:80]], axis=1)
        else:
            dn = dd
        h_o[...] = hh
        g_o[...] = hh * dn

    return pl.pallas_call(
        body,
        grid=(npk // rb,),
        in_specs=[
            pl.BlockSpec((rb, 128), lambda i: (i, 0)),
            pl.BlockSpec((rb, 128), lambda i: (i, 0)),
            pl.BlockSpec((2, rb, 128), lambda i: (0, i, 0)),
            pl.BlockSpec((1, 128), lambda i: (0, 0)),
            pl.BlockSpec((128, hout), lambda i: (0, 0)),
        ],
        out_specs=[
            pl.BlockSpec((rb, hout), lambda i: (i, 0)),
            pl.BlockSpec((rb, hout), lambda i: (i, 0)),
        ],
        out_shape=[
            jax.ShapeDtypeStruct((npk, hout), jnp.float32),
            jax.ShapeDtypeStruct((npk, hout), jnp.float32),
        ],
    )


def _make_final(npk, o, rb):
    def body(dis, h3, p, b, out_o):
        dn = jnp.concatenate([dis[:, 0:16], dis[:, 64:80]], axis=1)
        full = dn * (p[0] + p[1]) + (dn * dn) * h3[...] + b[...]
        outs = []
        for half in range(2):
            o2 = full[:, 16 * half : 16 * half + o]
            mx = jnp.max(o2, axis=1, keepdims=True)
            lse = mx + jnp.log(jnp.sum(jnp.exp(o2 - mx), axis=1, keepdims=True))
            outs.append(o2 - lse)
        out_o[...] = jnp.concatenate(outs, axis=1)

    return pl.pallas_call(
        body,
        grid=(npk // rb,),
        in_specs=[
            pl.BlockSpec((rb, 128), lambda i: (i, 0)),
            pl.BlockSpec((rb, 32), lambda i: (i, 0)),
            pl.BlockSpec((2, rb, 32), lambda i: (0, i, 0)),
            pl.BlockSpec((1, 32), lambda i: (0, 0)),
        ],
        out_specs=pl.BlockSpec((rb, 2 * o), lambda i: (i, 0)),
        out_shape=jax.ShapeDtypeStruct((npk, 2 * o), jnp.float32),
    )


def _blockdiag(w):
    z = jnp.zeros_like(w)
    return jnp.concatenate(
        [jnp.concatenate([w, z], axis=1), jnp.concatenate([z, w], axis=1)], axis=0
    )


def kernel(x, edge_index, W1, b1, W2, b2, W3, b3):
    n, d = x.shape
    hdim = W1.shape[1]
    o = W3.shape[1]
    e = edge_index.shape[1]
    npk = n // 2
    rb = 1000

    k = EDGE_K
    c_chunks = e // (NW * k)
    src3 = edge_index[0].reshape(NW, c_chunks, k)
    dst3 = edge_index[1].reshape(NW, c_chunks, k)

    w1d = _blockdiag(W1)
    w2d = _blockdiag(W2)
    w3p = jnp.zeros((hdim, LANES), jnp.float32).at[:, :o].set(W3)
    w3d = _blockdiag(w3p)
    b1p = jnp.concatenate([b1, b1]).reshape(1, 2 * hdim)
    b2p = jnp.concatenate([b2, b2]).reshape(1, 2 * hdim)
    b3pad = jnp.zeros((LANES,), jnp.float32).at[:o].set(b3)
    b3p = jnp.concatenate([b3pad, b3pad]).reshape(1, 2 * LANES)

    xpair = x.reshape(npk, 2 * d)

    sc16 = _make_sc_scatter(n, LANES, e, nbuf=10)
    sc64 = _make_sc_scatter(n, hdim, e, nbuf=8)

    deg64 = _make_sc_degree(n, e)(dst3)
    h1p = _make_a0(npk, 2 * d, rb)(xpair, w1d)
    dis64p, g1p = _make_a1(npk, rb)(deg64.reshape(2, npk, 128), h1p)
    p1 = sc64(g1p.reshape(n, hdim), src3, dst3)
    h2p, g2p = _make_combine_matmul(npk, 128, rb, False)(
        dis64p, h1p, p1.reshape(2, npk, 128), b1p, w2d
    )
    p2 = sc64(g2p.reshape(n, hdim), src3, dst3)
    h3pk, g3pk = _make_combine_matmul(npk, 32, rb, True)(
        dis64p, h2p, p2.reshape(2, npk, 128), b2p, w3d
    )
    p3 = sc16(g3pk.reshape(n, LANES), src3, dst3)
    outp = _make_final(npk, o, rb)(dis64p, h3pk, p3.reshape(2, npk, 32), b3p)
    return outp.reshape(n, o)
